# Initial kernel scaffold; baseline (speedup 1.0000x reference)
import jax
import jax.numpy as jnp
from jax.experimental import pallas as pl


def kernel(features, segment_ids):
    raise NotImplementedError("write your pallas kernel here")


# speedup vs baseline: 1.7726x; 1.7726x over previous
import jax
import jax.numpy as jnp
from jax import lax
from jax.experimental import pallas as pl
from jax.experimental.pallas import tpu as pltpu
from jax.experimental.pallas import tpu_sc as plsc

N_NODES = 100000
P = 4
D = 128
W = P * D
S = 512
NC = 2
NS = 16
NW = NC * NS
NR = 8
NQ = 4
CW = W // NQ
G8 = N_NODES // 8
BLK = 128
NFULL = 97
NTAIL = 11
IDS_PAD = 12520
SPAD = S + 16


def _sc_body(f_hbm, ids_hbm, outp, outc, acc_v, rows_v, rows8, ids_l, cnt_v):
    cid = lax.axis_index("c")
    sid = lax.axis_index("s")
    wid = cid * NS + sid
    r = wid // NQ
    q = wid % NQ
    base = ((r * G8) // NR) * 8
    end = (((r + 1) * G8) // NR) * 8

    zeros16 = jnp.zeros((16,), jnp.float32)

    def zrow(i, carry):
        for c in range(CW // 16):
            acc_v[i, pl.ds(c * 16, 16)] = zeros16
        return carry

    lax.fori_loop(0, S, zrow, 0)
    for z in range(SPAD // 16):
        cnt_v[pl.ds(z * 16, 16)] = zeros16

    pltpu.sync_copy(ids_hbm.at[pl.ds(base, 12504)], ids_l.at[pl.ds(0, 12504)])

    e0 = jnp.where(lax.iota(jnp.int32, 16) == 0, 1.0, 0.0)
    count_q = q == 0

    @pl.when(count_q)
    def _():
        n = end - base

        def cblk(i, carry):
            chunk = ids_l[pl.ds(i * 16, 16)]
            for l in range(16):
                seg = chunk[l]
                cnt_v[pl.ds(seg, 16)] = cnt_v[pl.ds(seg, 16)] + e0
            return carry

        lax.fori_loop(0, n // 16, cblk, 0)

        @pl.when(n % 16 == 8)
        def _():
            chunk = ids_l[pl.ds((n // 16) * 16, 16)]
            for l in range(8):
                seg = chunk[l]
                cnt_v[pl.ds(seg, 16)] = cnt_v[pl.ds(seg, 16)] + e0

    def accum16(loff, roff, nvalid):
        chunk = ids_l[pl.ds(loff, 16)]
        for l in range(nvalid):
            seg = chunk[l]
            for c in range(CW // 16):
                sl = pl.ds(c * 16, 16)
                plsc.addupdate(acc_v.at[seg, sl], rows_v[roff + l, sl])

    def blk(i, carry):
        off = base + i * BLK
        pltpu.sync_copy(f_hbm.at[pl.ds(off, BLK), pl.ds(q * CW, CW)], rows_v)

        def grp(g, c2):
            accum16(i * BLK + g * 16, g * 16, 16)
            return c2

        lax.fori_loop(0, BLK // 16, grp, 0)
        return carry

    lax.fori_loop(0, NFULL, blk, 0)

    t0 = base + NFULL * BLK
    for j in range(NTAIL):
        @pl.when(t0 + j * 8 < end)
        def _():
            off = t0 + j * 8
            pltpu.sync_copy(f_hbm.at[pl.ds(off, 8), pl.ds(q * CW, CW)],
                            rows8)
            chunk = ids_l[pl.ds(NFULL * BLK + j * 8, 16)]
            for l in range(8):
                seg = chunk[l]
                for c in range(CW // 16):
                    sl = pl.ds(c * 16, 16)
                    plsc.addupdate(acc_v.at[seg, sl], rows8[l, sl])

    pltpu.sync_copy(acc_v, outp.at[r, q])
    @pl.when(count_q)
    def _():
        pltpu.sync_copy(cnt_v.at[pl.ds(0, S)], outc.at[r])


_sc_pool = pl.kernel(
    _sc_body,
    out_type=(jax.ShapeDtypeStruct((NR, NQ, S, CW), jnp.float32),
              jax.ShapeDtypeStruct((NR, S), jnp.float32)),
    mesh=plsc.VectorSubcoreMesh(core_axis_name="c", subcore_axis_name="s"),
    scratch_types=[
        pltpu.VMEM((S, CW), jnp.float32),
        pltpu.VMEM((BLK, CW), jnp.float32),
        pltpu.VMEM((8, CW), jnp.float32),
        pltpu.VMEM((IDS_PAD,), jnp.int32),
        pltpu.VMEM((SPAD,), jnp.float32),
    ],
)


def _fin_body(p_ref, c_ref, o_ref):
    sums = jnp.sum(p_ref[:, 0], axis=0)
    counts = jnp.sum(c_ref[...], axis=0)
    o_ref[...] = sums / jnp.maximum(counts, 1.0)[:, None]


_finalize = pl.pallas_call(
    _fin_body,
    grid=(NQ,),
    in_specs=[
        pl.BlockSpec((NR, 1, S, CW), lambda i: (0, i, 0, 0)),
        pl.BlockSpec((NR, S), lambda i: (0, 0)),
    ],
    out_specs=pl.BlockSpec((S, CW), lambda i: (0, i)),
    out_shape=jax.ShapeDtypeStruct((S, W), jnp.float32),
)


@jax.jit
def kernel(features, segment_ids):
    f2 = features.reshape(N_NODES, W)
    ids = segment_ids.astype(jnp.int32)
    partials, counts = _sc_pool(f2, ids)
    pooled = _finalize(partials, counts)
    return pooled.reshape(S, P, D).transpose(0, 2, 1).reshape(S, W)


# speedup vs baseline: 2.4102x; 1.3597x over previous
import jax
import jax.numpy as jnp
from jax import lax
from jax.experimental import pallas as pl
from jax.experimental.pallas import tpu as pltpu
from jax.experimental.pallas import tpu_sc as plsc

N_NODES = 100000
P = 4
D = 128
W = P * D
S = 512
NC = 2
NS = 16
NW = NC * NS
NR = 8
NQ = 4
CW = W // NQ
G8 = N_NODES // 8
BLK = 128
NFULL = 97
NTAIL = 11
IDS_PAD = 12520
SPAD = S + 16


def _sc_body(f_hbm, ids_hbm, outp, outc, acc_v, rows_v, rows8, ids_l, cnt_v,
             rsum_v):
    cid = lax.axis_index("c")
    sid = lax.axis_index("s")
    wid = cid * NS + sid
    r = wid // NQ
    q = wid % NQ
    base = ((r * G8) // NR) * 8
    end = (((r + 1) * G8) // NR) * 8

    zeros16 = jnp.zeros((16,), jnp.float32)

    def zrow(i, carry):
        for c in range(CW // 16):
            acc_v[i, pl.ds(c * 16, 16)] = zeros16
        return carry

    lax.fori_loop(0, S, zrow, 0)
    for z in range(SPAD // 16):
        cnt_v[pl.ds(z * 16, 16)] = zeros16
    for z in range(CW // 16):
        rsum_v[pl.ds(z * 16, 16)] = zeros16

    pltpu.sync_copy(ids_hbm.at[pl.ds(base, 12504)], ids_l.at[pl.ds(0, 12504)])

    e0 = jnp.where(lax.iota(jnp.int32, 16) == 0, 1.0, 0.0)
    count_q = q == 0
    n = end - base

    @pl.when(count_q)
    def _():
        def cblk(i, carry):
            chunk = ids_l[pl.ds(i * 16, 16)]
            s0 = chunk[0]
            uniform = s0 == chunk[15]

            @pl.when(uniform)
            def _():
                cnt_v[pl.ds(s0, 16)] = cnt_v[pl.ds(s0, 16)] + 16.0 * e0

            @pl.when(jnp.logical_not(uniform))
            def _():
                for l in range(16):
                    seg = chunk[l]
                    cnt_v[pl.ds(seg, 16)] = cnt_v[pl.ds(seg, 16)] + e0
            return carry

        lax.fori_loop(0, n // 16, cblk, 0)

        @pl.when(n % 16 == 8)
        def _():
            chunk = ids_l[pl.ds((n // 16) * 16, 16)]
            for l in range(8):
                seg = chunk[l]
                cnt_v[pl.ds(seg, 16)] = cnt_v[pl.ds(seg, 16)] + e0

    NCH = CW // 16
    zreg = jnp.zeros((16,), jnp.float32)

    cur0 = ids_l[pl.ds(0, 16)][0]

    def blk(i, cur):
        off = base + i * BLK
        pltpu.sync_copy(f_hbm.at[pl.ds(off, BLK), pl.ds(q * CW, CW)], rows_v)

        def grp(g, cur_g):
            chunk = ids_l[pl.ds(i * BLK + g * 16, 16)]
            s0 = chunk[0]
            s15 = chunk[15]
            uniform = s0 == s15

            csum = [rows_v[g * 16, pl.ds(c * 16, 16)] for c in range(NCH)]
            for l in range(1, 16):
                for c in range(NCH):
                    csum[c] = csum[c] + rows_v[g * 16 + l, pl.ds(c * 16, 16)]

            need_flush = jnp.logical_or(jnp.logical_not(uniform),
                                        s0 != cur_g)

            @pl.when(need_flush)
            def _():
                for c in range(NCH):
                    sl = pl.ds(c * 16, 16)
                    plsc.addupdate(acc_v.at[cur_g, sl], rsum_v[sl])
                    rsum_v[sl] = zreg

            @pl.when(uniform)
            def _():
                for c in range(NCH):
                    sl = pl.ds(c * 16, 16)
                    plsc.addupdate(rsum_v.at[sl], csum[c])

            @pl.when(jnp.logical_not(uniform))
            def _():
                for l in range(16):
                    seg = chunk[l]
                    for c in range(NCH):
                        sl = pl.ds(c * 16, 16)
                        plsc.addupdate(acc_v.at[seg, sl],
                                       rows_v[g * 16 + l, sl])

            return jnp.where(uniform, s0, s15)

        return lax.fori_loop(0, BLK // 16, grp, cur)

    cur_end = lax.fori_loop(0, NFULL, blk, cur0)
    for c in range(NCH):
        sl = pl.ds(c * 16, 16)
        plsc.addupdate(acc_v.at[cur_end, sl], rsum_v[sl])

    t0 = base + NFULL * BLK
    for j in range(NTAIL):
        @pl.when(t0 + j * 8 < end)
        def _():
            off = t0 + j * 8
            pltpu.sync_copy(f_hbm.at[pl.ds(off, 8), pl.ds(q * CW, CW)],
                            rows8)
            chunk = ids_l[pl.ds(NFULL * BLK + j * 8, 16)]
            for l in range(8):
                seg = chunk[l]
                for c in range(NCH):
                    sl = pl.ds(c * 16, 16)
                    plsc.addupdate(acc_v.at[seg, sl], rows8[l, sl])

    pltpu.sync_copy(acc_v, outp.at[r, q])
    @pl.when(count_q)
    def _():
        pltpu.sync_copy(cnt_v.at[pl.ds(0, S)], outc.at[r])


_sc_pool = pl.kernel(
    _sc_body,
    out_type=(jax.ShapeDtypeStruct((NR, NQ, S, CW), jnp.float32),
              jax.ShapeDtypeStruct((NR, S), jnp.float32)),
    mesh=plsc.VectorSubcoreMesh(core_axis_name="c", subcore_axis_name="s"),
    scratch_types=[
        pltpu.VMEM((S, CW), jnp.float32),
        pltpu.VMEM((BLK, CW), jnp.float32),
        pltpu.VMEM((8, CW), jnp.float32),
        pltpu.VMEM((IDS_PAD,), jnp.int32),
        pltpu.VMEM((SPAD,), jnp.float32),
        pltpu.VMEM((CW,), jnp.float32),
    ],
)


def _fin_body(p_ref, c_ref, o_ref):
    sums = jnp.sum(p_ref[:, 0], axis=0)
    counts = jnp.sum(c_ref[...], axis=0)
    o_ref[...] = sums / jnp.maximum(counts, 1.0)[:, None]


_finalize = pl.pallas_call(
    _fin_body,
    grid=(NQ,),
    in_specs=[
        pl.BlockSpec((NR, 1, S, CW), lambda i: (0, i, 0, 0)),
        pl.BlockSpec((NR, S), lambda i: (0, 0)),
    ],
    out_specs=pl.BlockSpec((S, CW), lambda i: (0, i)),
    out_shape=jax.ShapeDtypeStruct((S, W), jnp.float32),
)


@jax.jit
def kernel(features, segment_ids):
    f2 = features.reshape(N_NODES, W)
    ids = segment_ids.astype(jnp.int32)
    partials, counts = _sc_pool(f2, ids)
    pooled = _finalize(partials, counts)
    return pooled.reshape(S, P, D).transpose(0, 2, 1).reshape(S, W)


# speedup vs baseline: 3.0830x; 1.2791x over previous
import jax
import jax.numpy as jnp
from jax import lax
from jax.experimental import pallas as pl
from jax.experimental.pallas import tpu as pltpu
from jax.experimental.pallas import tpu_sc as plsc

N_NODES = 100000
P = 4
D = 128
W = P * D
S = 512
NC = 2
NS = 16
NW = NC * NS
NR = 8
NQ = 4
CW = W // NQ
G8 = N_NODES // 8
BLK = 128
NFULL = 97
NTAIL = 11
IDS_PAD = 12520
SPAD = S + 16


def _sc_body(f_hbm, ids_hbm, outp, outc, acc_v, rows_v, rows8, ids_l, cnt_v,
             rsum_v, sem0, sem1):
    cid = lax.axis_index("c")
    sid = lax.axis_index("s")
    wid = cid * NS + sid
    r = wid // NQ
    q = wid % NQ
    base = ((r * G8) // NR) * 8
    end = (((r + 1) * G8) // NR) * 8

    zeros16 = jnp.zeros((16,), jnp.float32)

    def zrow(i, carry):
        for c in range(CW // 16):
            acc_v[i, pl.ds(c * 16, 16)] = zeros16
        return carry

    lax.fori_loop(0, S, zrow, 0)
    for z in range(SPAD // 16):
        cnt_v[pl.ds(z * 16, 16)] = zeros16
    for z in range(CW // 16):
        rsum_v[pl.ds(z * 16, 16)] = zeros16

    pltpu.sync_copy(ids_hbm.at[pl.ds(base, 12504)], ids_l.at[pl.ds(0, 12504)])

    e0 = jnp.where(lax.iota(jnp.int32, 16) == 0, 1.0, 0.0)
    count_q = q == 0
    n = end - base

    @pl.when(count_q)
    def _():
        def cblk(i, carry):
            chunk = ids_l[pl.ds(i * 16, 16)]
            s0 = chunk[0]
            uniform = s0 == chunk[15]

            @pl.when(uniform)
            def _():
                cnt_v[pl.ds(s0, 16)] = cnt_v[pl.ds(s0, 16)] + 16.0 * e0

            @pl.when(jnp.logical_not(uniform))
            def _():
                for l in range(16):
                    seg = chunk[l]
                    cnt_v[pl.ds(seg, 16)] = cnt_v[pl.ds(seg, 16)] + e0
            return carry

        lax.fori_loop(0, n // 16, cblk, 0)

        @pl.when(n % 16 == 8)
        def _():
            chunk = ids_l[pl.ds((n // 16) * 16, 16)]
            for l in range(8):
                seg = chunk[l]
                cnt_v[pl.ds(seg, 16)] = cnt_v[pl.ds(seg, 16)] + e0

    NCH = CW // 16
    zreg = jnp.zeros((16,), jnp.float32)

    cur0 = ids_l[pl.ds(0, 16)][0]
    sems = (sem0, sem1)

    def _start(i, b):
        off = base + i * BLK
        pltpu.async_copy(f_hbm.at[pl.ds(off, BLK), pl.ds(q * CW, CW)],
                         rows_v.at[b], sems[b])

    def _wait(i, b):
        off = base + i * BLK
        pltpu.make_async_copy(f_hbm.at[pl.ds(off, BLK), pl.ds(q * CW, CW)],
                              rows_v.at[b], sems[b]).wait()

    def _consume(i, b, cur):
        def grp(g, cur_g):
            chunk = ids_l[pl.ds(i * BLK + g * 16, 16)]
            s0 = chunk[0]
            s15 = chunk[15]
            uniform = s0 == s15

            csum = [rows_v[b, g * 16, pl.ds(c * 16, 16)] for c in range(NCH)]
            for l in range(1, 16):
                for c in range(NCH):
                    csum[c] = csum[c] + rows_v[b, g * 16 + l,
                                               pl.ds(c * 16, 16)]

            need_flush = jnp.logical_or(jnp.logical_not(uniform),
                                        s0 != cur_g)

            @pl.when(need_flush)
            def _():
                for c in range(NCH):
                    sl = pl.ds(c * 16, 16)
                    plsc.addupdate(acc_v.at[cur_g, sl], rsum_v[sl])
                    rsum_v[sl] = zreg

            @pl.when(uniform)
            def _():
                for c in range(NCH):
                    sl = pl.ds(c * 16, 16)
                    plsc.addupdate(rsum_v.at[sl], csum[c])

            @pl.when(jnp.logical_not(uniform))
            def _():
                for l in range(16):
                    seg = chunk[l]
                    for c in range(NCH):
                        sl = pl.ds(c * 16, 16)
                        plsc.addupdate(acc_v.at[seg, sl],
                                       rows_v[b, g * 16 + l, sl])

            return jnp.where(uniform, s0, s15)

        return lax.fori_loop(0, BLK // 16, grp, cur)

    _start(0, 0)
    _start(1, 1)

    def pair(g, cur):
        i0 = 2 * g
        for b in range(2):
            i = i0 + b
            _wait(i, b)
            cur = _consume(i, b, cur)

            @pl.when(i + 2 < NFULL)
            def _():
                _start(i + 2, b)
        return cur

    cur_l = lax.fori_loop(0, (NFULL - 1) // 2, pair, cur0)
    _wait(NFULL - 1, 0)
    cur_end = _consume(NFULL - 1, 0, cur_l)
    for c in range(NCH):
        sl = pl.ds(c * 16, 16)
        plsc.addupdate(acc_v.at[cur_end, sl], rsum_v[sl])

    t0 = base + NFULL * BLK
    for j in range(NTAIL):
        @pl.when(t0 + j * 8 < end)
        def _():
            off = t0 + j * 8
            pltpu.sync_copy(f_hbm.at[pl.ds(off, 8), pl.ds(q * CW, CW)],
                            rows8)
            chunk = ids_l[pl.ds(NFULL * BLK + j * 8, 16)]
            for l in range(8):
                seg = chunk[l]
                for c in range(NCH):
                    sl = pl.ds(c * 16, 16)
                    plsc.addupdate(acc_v.at[seg, sl], rows8[l, sl])

    pltpu.sync_copy(acc_v, outp.at[r, q])
    @pl.when(count_q)
    def _():
        pltpu.sync_copy(cnt_v.at[pl.ds(0, S)], outc.at[r])


_sc_pool = pl.kernel(
    _sc_body,
    out_type=(jax.ShapeDtypeStruct((NR, NQ, S, CW), jnp.float32),
              jax.ShapeDtypeStruct((NR, S), jnp.float32)),
    mesh=plsc.VectorSubcoreMesh(core_axis_name="c", subcore_axis_name="s"),
    scratch_types=[
        pltpu.VMEM((S, CW), jnp.float32),
        pltpu.VMEM((2, BLK, CW), jnp.float32),
        pltpu.VMEM((8, CW), jnp.float32),
        pltpu.VMEM((IDS_PAD,), jnp.int32),
        pltpu.VMEM((SPAD,), jnp.float32),
        pltpu.VMEM((CW,), jnp.float32),
        pltpu.SemaphoreType.DMA,
        pltpu.SemaphoreType.DMA,
    ],
)


def _fin_body(p_ref, c_ref, o_ref):
    sums = jnp.sum(p_ref[:, 0], axis=0)
    counts = jnp.sum(c_ref[...], axis=0)
    o_ref[...] = sums / jnp.maximum(counts, 1.0)[:, None]


_finalize = pl.pallas_call(
    _fin_body,
    grid=(NQ,),
    in_specs=[
        pl.BlockSpec((NR, 1, S, CW), lambda i: (0, i, 0, 0)),
        pl.BlockSpec((NR, S), lambda i: (0, 0)),
    ],
    out_specs=pl.BlockSpec((S, CW), lambda i: (0, i)),
    out_shape=jax.ShapeDtypeStruct((S, W), jnp.float32),
)


@jax.jit
def kernel(features, segment_ids):
    f2 = features.reshape(N_NODES, W)
    ids = segment_ids.astype(jnp.int32)
    partials, counts = _sc_pool(f2, ids)
    pooled = _finalize(partials, counts)
    return pooled.reshape(S, P, D).transpose(0, 2, 1).reshape(S, W)


# speedup vs baseline: 4.7366x; 1.5364x over previous
import jax
import jax.numpy as jnp
from jax import lax
from jax.experimental import pallas as pl
from jax.experimental.pallas import tpu as pltpu
from jax.experimental.pallas import tpu_sc as plsc

N_NODES = 100000
P = 4
D = 128
W = P * D
S = 512
NC = 2
NS = 16
NW = NC * NS
NR = 8
NQ = 4
CW = W // NQ
G8 = N_NODES // 8
BLK = 128
NFULL = 97
NTAIL = 11
IDS_PAD = 12520
SPAD = S + 16


def _sc_body(f_hbm, ids_hbm, outp, outc, acc_v, rows_v, rows8, ids_l, cnt_v,
             rsum_v, sem0, sem1):
    cid = lax.axis_index("c")
    sid = lax.axis_index("s")
    wid = cid * NS + sid
    r = wid // NQ
    q = wid % NQ
    base = ((r * G8) // NR) * 8
    end = (((r + 1) * G8) // NR) * 8

    zeros16 = jnp.zeros((16,), jnp.float32)

    def zrow(i, carry):
        for c in range(CW // 16):
            acc_v[i, pl.ds(c * 16, 16)] = zeros16
        return carry

    lax.fori_loop(0, S, zrow, 0)
    for z in range(SPAD // 16):
        cnt_v[pl.ds(z * 16, 16)] = zeros16
    for z in range(CW // 16):
        rsum_v[pl.ds(z * 16, 16)] = zeros16

    pltpu.sync_copy(ids_hbm.at[pl.ds(base, 12504)], ids_l.at[pl.ds(0, 12504)])

    e0 = jnp.where(lax.iota(jnp.int32, 16) == 0, 1.0, 0.0)
    count_q = q == 0
    n = end - base

    sems = (sem0, sem1)

    def _start(i, b):
        off = base + i * BLK
        pltpu.async_copy(f_hbm.at[pl.ds(off, BLK), q],
                         rows_v.at[b], sems[b])

    def _wait(i, b):
        off = base + i * BLK
        pltpu.make_async_copy(f_hbm.at[pl.ds(off, BLK), q],
                              rows_v.at[b], sems[b]).wait()

    _start(0, 0)
    _start(1, 1)

    @pl.when(count_q)
    def _():
        def cblk(i, carry):
            chunk = ids_l[pl.ds(i * 16, 16)]
            s0 = chunk[0]
            uniform = s0 == chunk[15]

            @pl.when(uniform)
            def _():
                cnt_v[pl.ds(s0, 16)] = cnt_v[pl.ds(s0, 16)] + 16.0 * e0

            @pl.when(jnp.logical_not(uniform))
            def _():
                for l in range(16):
                    seg = chunk[l]
                    cnt_v[pl.ds(seg, 16)] = cnt_v[pl.ds(seg, 16)] + e0
            return carry

        lax.fori_loop(0, n // 16, cblk, 0)

        @pl.when(n % 16 == 8)
        def _():
            chunk = ids_l[pl.ds((n // 16) * 16, 16)]
            for l in range(8):
                seg = chunk[l]
                cnt_v[pl.ds(seg, 16)] = cnt_v[pl.ds(seg, 16)] + e0

    NCH = CW // 16
    zreg = jnp.zeros((16,), jnp.float32)

    cur0 = ids_l[pl.ds(0, 16)][0]

    def _consume(i, b, cur):
        def grp(g, cur_g):
            chunk = ids_l[pl.ds(i * BLK + g * 16, 16)]
            s0 = chunk[0]
            s15 = chunk[15]
            uniform = s0 == s15

            csum = [rows_v[b, g * 16, pl.ds(c * 16, 16)] for c in range(NCH)]
            for l in range(1, 16):
                for c in range(NCH):
                    csum[c] = csum[c] + rows_v[b, g * 16 + l,
                                               pl.ds(c * 16, 16)]

            need_flush = jnp.logical_or(jnp.logical_not(uniform),
                                        s0 != cur_g)

            @pl.when(need_flush)
            def _():
                for c in range(NCH):
                    sl = pl.ds(c * 16, 16)
                    plsc.addupdate(acc_v.at[cur_g, sl], rsum_v[sl])
                    rsum_v[sl] = zreg

            @pl.when(uniform)
            def _():
                for c in range(NCH):
                    sl = pl.ds(c * 16, 16)
                    plsc.addupdate(rsum_v.at[sl], csum[c])

            @pl.when(jnp.logical_not(uniform))
            def _():
                for l in range(16):
                    seg = chunk[l]
                    for c in range(NCH):
                        sl = pl.ds(c * 16, 16)
                        plsc.addupdate(acc_v.at[seg, sl],
                                       rows_v[b, g * 16 + l, sl])

            return jnp.where(uniform, s0, s15)

        return lax.fori_loop(0, BLK // 16, grp, cur)

    def pair(g, cur):
        i0 = 2 * g
        for b in range(2):
            i = i0 + b
            _wait(i, b)
            cur = _consume(i, b, cur)

            @pl.when(i + 2 < NFULL)
            def _():
                _start(i + 2, b)
        return cur

    cur_l = lax.fori_loop(0, (NFULL - 1) // 2, pair, cur0)
    _wait(NFULL - 1, 0)
    cur_end = _consume(NFULL - 1, 0, cur_l)
    for c in range(NCH):
        sl = pl.ds(c * 16, 16)
        plsc.addupdate(acc_v.at[cur_end, sl], rsum_v[sl])

    t0 = base + NFULL * BLK
    for j in range(NTAIL):
        @pl.when(t0 + j * 8 < end)
        def _():
            off = t0 + j * 8
            pltpu.sync_copy(f_hbm.at[pl.ds(off, 8), q], rows8)
            chunk = ids_l[pl.ds(NFULL * BLK + j * 8, 16)]
            for l in range(8):
                seg = chunk[l]
                for c in range(NCH):
                    sl = pl.ds(c * 16, 16)
                    plsc.addupdate(acc_v.at[seg, sl], rows8[l, sl])

    pltpu.sync_copy(acc_v, outp.at[r, q])
    @pl.when(count_q)
    def _():
        pltpu.sync_copy(cnt_v.at[pl.ds(0, S)], outc.at[r])


_sc_pool = pl.kernel(
    _sc_body,
    out_type=(jax.ShapeDtypeStruct((NR, NQ, S, CW), jnp.float32),
              jax.ShapeDtypeStruct((NR, S), jnp.float32)),
    mesh=plsc.VectorSubcoreMesh(core_axis_name="c", subcore_axis_name="s"),
    scratch_types=[
        pltpu.VMEM((S, CW), jnp.float32),
        pltpu.VMEM((2, BLK, CW), jnp.float32),
        pltpu.VMEM((8, CW), jnp.float32),
        pltpu.VMEM((IDS_PAD,), jnp.int32),
        pltpu.VMEM((SPAD,), jnp.float32),
        pltpu.VMEM((CW,), jnp.float32),
        pltpu.SemaphoreType.DMA,
        pltpu.SemaphoreType.DMA,
    ],
)


def _fin_body(p_ref, c_ref, o_ref):
    sums = jnp.sum(p_ref[:, 0], axis=0)
    counts = jnp.sum(c_ref[...], axis=0)
    o_ref[...] = sums / jnp.maximum(counts, 1.0)[:, None]


_finalize = pl.pallas_call(
    _fin_body,
    grid=(NQ,),
    in_specs=[
        pl.BlockSpec((NR, 1, S, CW), lambda i: (0, i, 0, 0)),
        pl.BlockSpec((NR, S), lambda i: (0, 0)),
    ],
    out_specs=pl.BlockSpec((S, CW), lambda i: (0, i)),
    out_shape=jax.ShapeDtypeStruct((S, W), jnp.float32),
)


@jax.jit
def kernel(features, segment_ids):
    ids = segment_ids.astype(jnp.int32)
    partials, counts = _sc_pool(features, ids)
    pooled = _finalize(partials, counts)
    return pooled.reshape(S, P, D).transpose(0, 2, 1).reshape(S, W)


# speedup vs baseline: 6.6409x; 1.4020x over previous
import jax
import jax.numpy as jnp
from jax import lax
from jax.experimental import pallas as pl
from jax.experimental.pallas import tpu as pltpu
from jax.experimental.pallas import tpu_sc as plsc

N_NODES = 100000
P = 4
D = 128
W = P * D
S = 512
NC = 2
NS = 16
NW = NC * NS
NR = 8
NQ = 4
CW = W // NQ
G8 = N_NODES // 8
BLK = 128
NFULL = 97
NTAIL = 11
IDS_PAD = 12520
SPAD = S + 16


def _sc_body(f_hbm, ids_hbm, outp, outc, acc_v, rows_v, rows8, ids_l, cnt_v,
             sem0, sem1):
    cid = lax.axis_index("c")
    sid = lax.axis_index("s")
    wid = cid * NS + sid
    r = wid // NQ
    q = wid % NQ
    base = ((r * G8) // NR) * 8
    end = (((r + 1) * G8) // NR) * 8

    zeros16 = jnp.zeros((16,), jnp.float32)

    def zrow(i, carry):
        for c in range(CW // 16):
            acc_v[i, pl.ds(c * 16, 16)] = zeros16
        return carry

    lax.fori_loop(0, S, zrow, 0)
    for z in range(SPAD // 16):
        cnt_v[pl.ds(z * 16, 16)] = zeros16

    pltpu.sync_copy(ids_hbm.at[pl.ds(base, 12504)], ids_l.at[pl.ds(0, 12504)])

    e0 = jnp.where(lax.iota(jnp.int32, 16) == 0, 1.0, 0.0)
    count_q = q == 0
    n = end - base

    sems = (sem0, sem1)

    def _start(i, b):
        off = base + i * BLK
        pltpu.async_copy(f_hbm.at[pl.ds(off, BLK), q],
                         rows_v.at[b], sems[b])

    def _wait(i, b):
        off = base + i * BLK
        pltpu.make_async_copy(f_hbm.at[pl.ds(off, BLK), q],
                              rows_v.at[b], sems[b]).wait()

    _start(0, 0)
    _start(1, 1)

    @pl.when(count_q)
    def _():
        def cblk(i, carry):
            chunk = ids_l[pl.ds(i * 16, 16)]
            s0 = chunk[0]
            uniform = s0 == chunk[15]

            @pl.when(uniform)
            def _():
                cnt_v[pl.ds(s0, 16)] = cnt_v[pl.ds(s0, 16)] + 16.0 * e0

            @pl.when(jnp.logical_not(uniform))
            def _():
                for l in range(16):
                    seg = chunk[l]
                    cnt_v[pl.ds(seg, 16)] = cnt_v[pl.ds(seg, 16)] + e0
            return carry

        lax.fori_loop(0, n // 16, cblk, 0)

        @pl.when(n % 16 == 8)
        def _():
            chunk = ids_l[pl.ds((n // 16) * 16, 16)]
            for l in range(8):
                seg = chunk[l]
                cnt_v[pl.ds(seg, 16)] = cnt_v[pl.ds(seg, 16)] + e0

    NCH = CW // 16

    def _consume(i, b):
        @plsc.parallel_loop(0, BLK // 16)
        def grp(g):
            chunk = ids_l[pl.ds(i * BLK + g * 16, 16)]
            s0 = chunk[0]
            s15 = chunk[15]
            uniform = s0 == s15

            csum = [rows_v[b, g * 16, pl.ds(c * 16, 16)] for c in range(NCH)]
            for l in range(1, 16):
                for c in range(NCH):
                    csum[c] = csum[c] + rows_v[b, g * 16 + l,
                                               pl.ds(c * 16, 16)]

            @pl.when(uniform)
            def _():
                for c in range(NCH):
                    sl = pl.ds(c * 16, 16)
                    plsc.addupdate(acc_v.at[s0, sl], csum[c])

            @pl.when(jnp.logical_not(uniform))
            def _():
                for l in range(16):
                    seg = chunk[l]
                    for c in range(NCH):
                        sl = pl.ds(c * 16, 16)
                        plsc.addupdate(acc_v.at[seg, sl],
                                       rows_v[b, g * 16 + l, sl])

    def pair(g, carry):
        i0 = 2 * g
        for b in range(2):
            i = i0 + b
            _wait(i, b)
            _consume(i, b)

            @pl.when(i + 2 < NFULL)
            def _():
                _start(i + 2, b)
        return carry

    lax.fori_loop(0, (NFULL - 1) // 2, pair, 0)
    _wait(NFULL - 1, 0)
    _consume(NFULL - 1, 0)

    t0 = base + NFULL * BLK
    for j in range(NTAIL):
        @pl.when(t0 + j * 8 < end)
        def _():
            off = t0 + j * 8
            pltpu.sync_copy(f_hbm.at[pl.ds(off, 8), q], rows8)
            chunk = ids_l[pl.ds(NFULL * BLK + j * 8, 16)]
            for l in range(8):
                seg = chunk[l]
                for c in range(NCH):
                    sl = pl.ds(c * 16, 16)
                    plsc.addupdate(acc_v.at[seg, sl], rows8[l, sl])

    pltpu.sync_copy(acc_v, outp.at[r, q])
    @pl.when(count_q)
    def _():
        pltpu.sync_copy(cnt_v.at[pl.ds(0, S)], outc.at[r])


_sc_pool = pl.kernel(
    _sc_body,
    out_type=(jax.ShapeDtypeStruct((NR, NQ, S, CW), jnp.float32),
              jax.ShapeDtypeStruct((NR, S), jnp.float32)),
    mesh=plsc.VectorSubcoreMesh(core_axis_name="c", subcore_axis_name="s"),
    scratch_types=[
        pltpu.VMEM((S, CW), jnp.float32),
        pltpu.VMEM((2, BLK, CW), jnp.float32),
        pltpu.VMEM((8, CW), jnp.float32),
        pltpu.VMEM((IDS_PAD,), jnp.int32),
        pltpu.VMEM((SPAD,), jnp.float32),
        pltpu.SemaphoreType.DMA,
        pltpu.SemaphoreType.DMA,
    ],
)


def _fin_body(p_ref, c_ref, o_ref):
    sums = jnp.sum(p_ref[:, 0], axis=0)
    counts = jnp.sum(c_ref[...], axis=0)
    o_ref[...] = sums / jnp.maximum(counts, 1.0)[:, None]


_finalize = pl.pallas_call(
    _fin_body,
    grid=(NQ,),
    in_specs=[
        pl.BlockSpec((NR, 1, S, CW), lambda i: (0, i, 0, 0)),
        pl.BlockSpec((NR, S), lambda i: (0, 0)),
    ],
    out_specs=pl.BlockSpec((S, CW), lambda i: (0, i)),
    out_shape=jax.ShapeDtypeStruct((S, W), jnp.float32),
)


@jax.jit
def kernel(features, segment_ids):
    ids = segment_ids.astype(jnp.int32)
    partials, counts = _sc_pool(features, ids)
    pooled = _finalize(partials, counts)
    return pooled.reshape(S, P, D).transpose(0, 2, 1).reshape(S, W)


# speedup vs baseline: 7.1593x; 1.0781x over previous
import jax
import jax.numpy as jnp
from jax import lax
from jax.experimental import pallas as pl
from jax.experimental.pallas import tpu as pltpu
from jax.experimental.pallas import tpu_sc as plsc

N_NODES = 100000
P = 4
D = 128
W = P * D
S = 512
NC = 2
NS = 16
NW = NC * NS
NR = 8
NQ = 4
CW = W // NQ
G8 = N_NODES // 8
BLK = 128
NFULL = 97
NTAIL = 11
IDS_PAD = 12520
SPAD = S + 16


def _sc_body(f_hbm, ids_hbm, outp, outc, acc_v, rows_v, rows8, ids_l, cnt_v,
             sem0, sem1):
    cid = lax.axis_index("c")
    sid = lax.axis_index("s")
    wid = cid * NS + sid
    r = wid // NQ
    q = wid % NQ
    base = ((r * G8) // NR) * 8
    end = (((r + 1) * G8) // NR) * 8

    zeros16 = jnp.zeros((16,), jnp.float32)

    @plsc.parallel_loop(0, S)
    def zrow(i):
        for c in range(CW // 16):
            acc_v[i, pl.ds(c * 16, 16)] = zeros16

    for z in range(SPAD // 16):
        cnt_v[pl.ds(z * 16, 16)] = zeros16

    pltpu.sync_copy(ids_hbm.at[pl.ds(base, 12504)], ids_l.at[pl.ds(0, 12504)])

    e0 = jnp.where(lax.iota(jnp.int32, 16) == 0, 1.0, 0.0)
    count_q = q == 0
    n = end - base

    sems = (sem0, sem1)

    def _start(i, b):
        off = base + i * BLK
        pltpu.async_copy(f_hbm.at[pl.ds(off, BLK), q],
                         rows_v.at[b], sems[b])

    def _wait(i, b):
        off = base + i * BLK
        pltpu.make_async_copy(f_hbm.at[pl.ds(off, BLK), q],
                              rows_v.at[b], sems[b]).wait()

    _start(0, 0)
    _start(1, 1)

    e16 = 16.0 * e0

    @pl.when(count_q)
    def _():
        @plsc.parallel_loop(0, n // 16)
        def cblk(i):
            chunk = ids_l[pl.ds(i * 16, 16)]
            s0 = chunk[0]
            uniform = s0 == chunk[15]

            @pl.when(uniform)
            def _():
                plsc.addupdate(cnt_v.at[pl.ds(s0, 16)], e16)

            @pl.when(jnp.logical_not(uniform))
            def _():
                for l in range(16):
                    seg = chunk[l]
                    plsc.addupdate(cnt_v.at[pl.ds(seg, 16)], e0)

        @pl.when(n % 16 == 8)
        def _():
            chunk = ids_l[pl.ds((n // 16) * 16, 16)]
            for l in range(8):
                seg = chunk[l]
                plsc.addupdate(cnt_v.at[pl.ds(seg, 16)], e0)

    NCH = CW // 16

    def _consume(i, b):
        @plsc.parallel_loop(0, BLK // 16)
        def grp(g):
            chunk = ids_l[pl.ds(i * BLK + g * 16, 16)]
            s0 = chunk[0]
            s15 = chunk[15]
            uniform = s0 == s15

            csum = [rows_v[b, g * 16, pl.ds(c * 16, 16)] for c in range(NCH)]
            for l in range(1, 16):
                for c in range(NCH):
                    csum[c] = csum[c] + rows_v[b, g * 16 + l,
                                               pl.ds(c * 16, 16)]

            @pl.when(uniform)
            def _():
                for c in range(NCH):
                    sl = pl.ds(c * 16, 16)
                    plsc.addupdate(acc_v.at[s0, sl], csum[c])

            @pl.when(jnp.logical_not(uniform))
            def _():
                for l in range(16):
                    seg = chunk[l]
                    for c in range(NCH):
                        sl = pl.ds(c * 16, 16)
                        plsc.addupdate(acc_v.at[seg, sl],
                                       rows_v[b, g * 16 + l, sl])

    def pair(g, carry):
        i0 = 2 * g
        for b in range(2):
            i = i0 + b
            _wait(i, b)
            _consume(i, b)

            @pl.when(i + 2 < NFULL)
            def _():
                _start(i + 2, b)
        return carry

    lax.fori_loop(0, (NFULL - 1) // 2, pair, 0)
    _wait(NFULL - 1, 0)
    _consume(NFULL - 1, 0)

    t0 = base + NFULL * BLK
    for j in range(NTAIL):
        @pl.when(t0 + j * 8 < end)
        def _():
            off = t0 + j * 8
            pltpu.sync_copy(f_hbm.at[pl.ds(off, 8), q], rows8)
            chunk = ids_l[pl.ds(NFULL * BLK + j * 8, 16)]
            for l in range(8):
                seg = chunk[l]
                for c in range(NCH):
                    sl = pl.ds(c * 16, 16)
                    plsc.addupdate(acc_v.at[seg, sl], rows8[l, sl])

    pltpu.sync_copy(acc_v, outp.at[r, q])
    @pl.when(count_q)
    def _():
        pltpu.sync_copy(cnt_v.at[pl.ds(0, S)], outc.at[r])


_sc_pool = pl.kernel(
    _sc_body,
    out_type=(jax.ShapeDtypeStruct((NR, NQ, S, CW), jnp.float32),
              jax.ShapeDtypeStruct((NR, S), jnp.float32)),
    mesh=plsc.VectorSubcoreMesh(core_axis_name="c", subcore_axis_name="s"),
    scratch_types=[
        pltpu.VMEM((S, CW), jnp.float32),
        pltpu.VMEM((2, BLK, CW), jnp.float32),
        pltpu.VMEM((8, CW), jnp.float32),
        pltpu.VMEM((IDS_PAD,), jnp.int32),
        pltpu.VMEM((SPAD,), jnp.float32),
        pltpu.SemaphoreType.DMA,
        pltpu.SemaphoreType.DMA,
    ],
)


def _fin_body(p_ref, c_ref, o_ref):
    sums = jnp.sum(p_ref[:, 0], axis=0)
    counts = jnp.sum(c_ref[...], axis=0)
    o_ref[...] = sums / jnp.maximum(counts, 1.0)[:, None]


_finalize = pl.pallas_call(
    _fin_body,
    grid=(NQ,),
    in_specs=[
        pl.BlockSpec((NR, 1, S, CW), lambda i: (0, i, 0, 0)),
        pl.BlockSpec((NR, S), lambda i: (0, 0)),
    ],
    out_specs=pl.BlockSpec((S, CW), lambda i: (0, i)),
    out_shape=jax.ShapeDtypeStruct((S, W), jnp.float32),
)


@jax.jit
def kernel(features, segment_ids):
    ids = segment_ids.astype(jnp.int32)
    partials, counts = _sc_pool(features, ids)
    pooled = _finalize(partials, counts)
    return pooled.reshape(S, P, D).transpose(0, 2, 1).reshape(S, W)
